# Initial kernel scaffold; baseline (speedup 1.0000x reference)
import jax
import jax.numpy as jnp
from jax.experimental import pallas as pl


def kernel(x, table, gamma, beta):
    raise NotImplementedError("write your pallas kernel here")


# speedup vs baseline: 3.3028x; 3.3028x over previous
import functools

import jax
import jax.numpy as jnp
from jax import lax
from jax.experimental import pallas as pl
from jax.experimental.pallas import tpu as pltpu
from jax.experimental.pallas import tpu_sc as plsc

EPS = 1e-5
D = 128


_ROWS_BLK = 2000


def _ln_body(tab_ref, gamma_ref, beta_ref, out_ref):
    t = tab_ref[...]
    mean = jnp.mean(t, axis=1, keepdims=True)
    c = t - mean
    var = jnp.mean(c * c, axis=1, keepdims=True)
    out_ref[...] = c * lax.rsqrt(var + EPS) * gamma_ref[...] + beta_ref[...]


def _ln_table(table, gamma, beta):
    n_rows = table.shape[0]
    grid = n_rows // _ROWS_BLK
    return pl.pallas_call(
        _ln_body,
        grid=(grid,),
        in_specs=[
            pl.BlockSpec((_ROWS_BLK, D), lambda i: (i, 0)),
            pl.BlockSpec((1, D), lambda i: (0, 0)),
            pl.BlockSpec((1, D), lambda i: (0, 0)),
        ],
        out_specs=pl.BlockSpec((_ROWS_BLK, D), lambda i: (i, 0)),
        out_shape=jax.ShapeDtypeStruct((n_rows, D), jnp.float32),
    )(table, gamma.reshape(1, D), beta.reshape(1, D))


_NC, _NS = 2, 16
_NW = _NC * _NS
_K = 640


def _make_gather(B):
    b_per_w = B // _NW
    nchunk = b_per_w // _K
    mesh = plsc.VectorSubcoreMesh(core_axis_name="c", subcore_axis_name="s")

    @functools.partial(
        pl.kernel,
        mesh=mesh,
        out_type=jax.ShapeDtypeStruct((B, D), jnp.float32),
        scratch_types=[
            pltpu.VMEM((b_per_w,), jnp.int32),
            pltpu.VMEM((_K, D), jnp.float32),
            pltpu.SemaphoreType.DMA,
        ],
    )
    def gather(tab_hbm, idx_hbm, out_hbm, idx_v, rows_v, sem):
        wid = lax.axis_index("s") * _NC + lax.axis_index("c")
        base = wid * b_per_w
        pltpu.sync_copy(idx_hbm.at[pl.ds(base, b_per_w)], idx_v)
        for g in range(nchunk):
            pltpu.async_copy(
                tab_hbm.at[idx_v.at[pl.ds(g * _K, _K)]], rows_v, sem
            ).wait()
            pltpu.sync_copy(rows_v, out_hbm.at[pl.ds(base + g * _K, _K)])

    return gather


def kernel(x, table, gamma, beta):
    batch, hist = x.shape
    B = batch * hist
    table_n = _ln_table(table, gamma, beta)
    out_flat = _make_gather(B)(table_n, x.reshape(B))
    return out_flat.reshape(batch, hist, D)


# speedup vs baseline: 4.1174x; 1.2466x over previous
import functools

import jax
import jax.numpy as jnp
from jax import lax
from jax.experimental import pallas as pl
from jax.experimental.pallas import tpu as pltpu
from jax.experimental.pallas import tpu_sc as plsc

EPS = 1e-5
D = 128


_NC, _NS = 2, 16
_NW = _NC * _NS
_K = 640


def _make_gather(B):
    b_per_w = B // _NW
    nchunk = b_per_w // _K
    mesh = plsc.VectorSubcoreMesh(core_axis_name="c", subcore_axis_name="s")

    @functools.partial(
        pl.kernel,
        mesh=mesh,
        out_type=jax.ShapeDtypeStruct((B, D), jnp.float32),
        scratch_types=[
            pltpu.VMEM((b_per_w,), jnp.int32),
            pltpu.VMEM((_K, D), jnp.float32),
            pltpu.SemaphoreType.DMA,
        ],
    )
    def gather(tab_hbm, idx_hbm, out_hbm, idx_v, rows_v, sem):
        wid = lax.axis_index("s") * _NC + lax.axis_index("c")
        base = wid * b_per_w
        pltpu.sync_copy(idx_hbm.at[pl.ds(base, b_per_w)], idx_v)
        for g in range(nchunk):
            pltpu.async_copy(
                tab_hbm.at[idx_v.at[pl.ds(g * _K, _K)]], rows_v, sem
            ).wait()
            pltpu.sync_copy(rows_v, out_hbm.at[pl.ds(base + g * _K, _K)])

    return gather


_BB = 64


def _ln_body(rows_ref, gamma_ref, beta_ref, out_ref):
    t = rows_ref[...]
    mean = jnp.mean(t, axis=1, keepdims=True)
    c = t - mean
    var = jnp.mean(c * c, axis=1, keepdims=True)
    normed = c * lax.rsqrt(var + EPS) * gamma_ref[...] + beta_ref[...]
    out_ref[...] = normed.reshape(out_ref.shape)


def _ln_out(rows_flat, gamma, beta, batch, hist):
    grid = batch // _BB
    return pl.pallas_call(
        _ln_body,
        grid=(grid,),
        in_specs=[
            pl.BlockSpec((_BB * hist, D), lambda i: (i, 0)),
            pl.BlockSpec((1, D), lambda i: (0, 0)),
            pl.BlockSpec((1, D), lambda i: (0, 0)),
        ],
        out_specs=pl.BlockSpec((_BB, hist, D), lambda i: (i, 0, 0)),
        out_shape=jax.ShapeDtypeStruct((batch, hist, D), jnp.float32),
    )(rows_flat, gamma.reshape(1, D), beta.reshape(1, D))


def kernel(x, table, gamma, beta):
    batch, hist = x.shape
    B = batch * hist
    rows_flat = _make_gather(B)(table, x.reshape(B))
    return _ln_out(rows_flat, gamma, beta, batch, hist)


# speedup vs baseline: 6.1038x; 1.4825x over previous
import functools

import jax
import jax.numpy as jnp
from jax import lax
from jax.experimental import pallas as pl
from jax.experimental.pallas import tpu as pltpu
from jax.experimental.pallas import tpu_sc as plsc

EPS = 1e-5
D = 128


_NC, _NS = 2, 16
_NW = _NC * _NS
_K = 640


def _make_gather(B):
    b_per_w = B // _NW
    nchunk = b_per_w // _K
    mesh = plsc.VectorSubcoreMesh(core_axis_name="c", subcore_axis_name="s")

    @functools.partial(
        pl.kernel,
        mesh=mesh,
        out_type=jax.ShapeDtypeStruct((B, D), jnp.float32),
        scratch_types=[
            pltpu.VMEM((b_per_w,), jnp.int32),
            pltpu.VMEM((_K, D), jnp.float32),
            pltpu.SemaphoreType.DMA,
        ],
    )
    def gather(tab_hbm, idx_hbm, out_hbm, idx_v, rows_v, sem):
        wid = lax.axis_index("s") * _NC + lax.axis_index("c")
        base = wid * b_per_w
        pltpu.sync_copy(idx_hbm.at[pl.ds(base, b_per_w)], idx_v)
        for g in range(nchunk):
            pltpu.async_copy(
                tab_hbm.at[idx_v.at[pl.ds(g * _K, _K)]], rows_v, sem
            ).wait()
            pltpu.sync_copy(rows_v, out_hbm.at[pl.ds(base + g * _K, _K)])

    return gather


def _ln_body(rows_ref, gamma_ref, beta_ref, out_ref):
    t = rows_ref[...]
    mean = jnp.mean(t, axis=1, keepdims=True)
    c = t - mean
    var = jnp.mean(c * c, axis=1, keepdims=True)
    normed = c * lax.rsqrt(var + EPS) * gamma_ref[...] + beta_ref[...]
    out_ref[...] = normed.reshape(out_ref.shape)


def _ln_out(rows_flat, gamma, beta, batch, hist):
    return pl.pallas_call(
        _ln_body,
        grid=(hist,),
        in_specs=[
            pl.BlockSpec((batch, D), lambda i: (i, 0)),
            pl.BlockSpec((1, D), lambda i: (0, 0)),
            pl.BlockSpec((1, D), lambda i: (0, 0)),
        ],
        out_specs=pl.BlockSpec((1, batch, D), lambda i: (i, 0, 0)),
        out_shape=jax.ShapeDtypeStruct((hist, batch, D), jnp.float32),
    )(rows_flat, gamma.reshape(1, D), beta.reshape(1, D))


def kernel(x, table, gamma, beta):
    batch, hist = x.shape
    B = batch * hist
    idx_lmajor = x.T.reshape(B)
    rows_flat = _make_gather(B)(table, idx_lmajor)
    out_t = _ln_out(rows_flat, gamma, beta, batch, hist)
    return out_t.transpose(1, 0, 2)


# speedup vs baseline: 6.6245x; 1.0853x over previous
import functools

import jax
import jax.numpy as jnp
from jax import lax
from jax.experimental import pallas as pl
from jax.experimental.pallas import tpu as pltpu
from jax.experimental.pallas import tpu_sc as plsc

EPS = 1e-5
D = 128
_SEG = 5


_NC, _NS = 2, 16
_NW = _NC * _NS
_K = 320


def _make_gather(B):
    b_per_w = B // _NW
    nchunk = b_per_w // _K
    mesh = plsc.VectorSubcoreMesh(core_axis_name="c", subcore_axis_name="s")

    @functools.partial(
        pl.kernel,
        mesh=mesh,
        out_type=jax.ShapeDtypeStruct((B, D), jnp.float32),
        scratch_types=[
            pltpu.VMEM((b_per_w,), jnp.int32),
            pltpu.VMEM((_K, D), jnp.float32),
            pltpu.VMEM((_K, D), jnp.float32),
            pltpu.SemaphoreType.DMA,
            pltpu.SemaphoreType.DMA,
            pltpu.SemaphoreType.DMA,
            pltpu.SemaphoreType.DMA,
        ],
    )
    def gather(tab_hbm, idx_hbm, out_hbm, idx_v, rows0, rows1, gs0, gs1, os0, os1):
        wid = lax.axis_index("s") * _NC + lax.axis_index("c")
        base = wid * b_per_w
        pltpu.sync_copy(idx_hbm.at[pl.ds(base, b_per_w)], idx_v)
        bufs = (rows0, rows1)
        gsems = (gs0, gs1)
        osems = (os0, os1)
        h_in = [None, None]
        h_out = [None, None]
        h_in[0] = pltpu.async_copy(
            tab_hbm.at[idx_v.at[pl.ds(0, _K)]], bufs[0], gsems[0]
        )
        for g in range(nchunk):
            b = g % 2
            if g + 1 < nchunk:
                b2 = (g + 1) % 2
                if h_out[b2] is not None:
                    h_out[b2].wait()
                h_in[b2] = pltpu.async_copy(
                    tab_hbm.at[idx_v.at[pl.ds((g + 1) * _K, _K)]],
                    bufs[b2],
                    gsems[b2],
                )
            h_in[b].wait()
            h_out[b] = pltpu.async_copy(
                bufs[b], out_hbm.at[pl.ds(base + g * _K, _K)], osems[b]
            )
        for h in h_out:
            if h is not None:
                h.wait()

    return gather


_INV_D = 1.0 / D


def _ln_compute(t, gamma, beta):
    mean = jnp.sum(t, axis=1, keepdims=True) * _INV_D
    m2 = jnp.sum(t * t, axis=1, keepdims=True) * _INV_D
    var = m2 - mean * mean
    return (t - mean) * lax.rsqrt(var + EPS) * gamma + beta


def _ln_body(rows_ref, gamma_ref, beta_ref, out_ref):
    normed = _ln_compute(rows_ref[...], gamma_ref[...], beta_ref[...])
    out_ref[...] = normed.reshape(out_ref.shape)


def _ln_seg_body(prev_ref, rows_ref, gamma_ref, beta_ref, out_ref):
    del prev_ref
    normed = _ln_compute(rows_ref[...], gamma_ref[...], beta_ref[...])
    out_ref[...] = normed.reshape(out_ref.shape)


def _ln_first(rows_seg, gamma, beta, batch, hist, hseg):
    return pl.pallas_call(
        _ln_body,
        grid=(hseg,),
        in_specs=[
            pl.BlockSpec((batch, D), lambda i: (i, 0)),
            pl.BlockSpec((1, D), lambda i: (0, 0)),
            pl.BlockSpec((1, D), lambda i: (0, 0)),
        ],
        out_specs=pl.BlockSpec((1, batch, D), lambda i: (i, 0, 0)),
        out_shape=jax.ShapeDtypeStruct((hist, batch, D), jnp.float32),
    )(rows_seg, gamma.reshape(1, D), beta.reshape(1, D))


def _ln_next(prev_full, rows_seg, gamma, beta, batch, hist, hseg, seg_base):
    return pl.pallas_call(
        _ln_seg_body,
        grid=(hseg,),
        in_specs=[
            pl.BlockSpec(memory_space=pl.ANY),
            pl.BlockSpec((batch, D), lambda i: (i, 0)),
            pl.BlockSpec((1, D), lambda i: (0, 0)),
            pl.BlockSpec((1, D), lambda i: (0, 0)),
        ],
        out_specs=pl.BlockSpec(
            (1, batch, D), lambda i: (seg_base + i, 0, 0)
        ),
        out_shape=jax.ShapeDtypeStruct((hist, batch, D), jnp.float32),
        input_output_aliases={0: 0},
    )(prev_full, rows_seg, gamma.reshape(1, D), beta.reshape(1, D))


def kernel(x, table, gamma, beta):
    batch, hist = x.shape
    B = batch * hist
    hseg = hist // _SEG
    bseg = batch * hseg
    idx_lmajor = x.T.reshape(B)
    sc_gather = _make_gather(bseg)
    rows = [
        sc_gather(table, lax.slice(idx_lmajor, (s * bseg,), ((s + 1) * bseg,)))
        for s in range(_SEG)
    ]
    out = _ln_first(rows[0], gamma, beta, batch, hist, hseg)
    for s in range(1, _SEG):
        out = _ln_next(out, rows[s], gamma, beta, batch, hist, hseg, s * hseg)
    return out.transpose(1, 0, 2)


# speedup vs baseline: 7.5427x; 1.1386x over previous
import functools

import jax
import jax.numpy as jnp
from jax import lax
from jax.experimental import pallas as pl
from jax.experimental.pallas import tpu as pltpu
from jax.experimental.pallas import tpu_sc as plsc

EPS = 1e-5
D = 128
_INV_D = 1.0 / D


_ROWS_BLK = 2000


def _ln_body(tab_ref, gamma_ref, beta_ref, out_ref):
    t = tab_ref[...]
    mean = jnp.sum(t, axis=1, keepdims=True) * _INV_D
    m2 = jnp.sum(t * t, axis=1, keepdims=True) * _INV_D
    var = m2 - mean * mean
    out_ref[...] = (t - mean) * lax.rsqrt(var + EPS) * gamma_ref[...] + beta_ref[...]


def _ln_table(table, gamma, beta):
    n_rows = table.shape[0]
    grid = n_rows // _ROWS_BLK
    return pl.pallas_call(
        _ln_body,
        grid=(grid,),
        in_specs=[
            pl.BlockSpec((_ROWS_BLK, D), lambda i: (i, 0)),
            pl.BlockSpec((1, D), lambda i: (0, 0)),
            pl.BlockSpec((1, D), lambda i: (0, 0)),
        ],
        out_specs=pl.BlockSpec((_ROWS_BLK, D), lambda i: (i, 0)),
        out_shape=jax.ShapeDtypeStruct((n_rows, D), jnp.float32),
    )(table, gamma.reshape(1, D), beta.reshape(1, D))


_NC, _NS = 2, 16
_NW = _NC * _NS
_K = 400


def _make_gather(B):
    b_per_w = B // _NW
    nchunk = b_per_w // _K
    mesh = plsc.VectorSubcoreMesh(core_axis_name="c", subcore_axis_name="s")

    @functools.partial(
        pl.kernel,
        mesh=mesh,
        out_type=jax.ShapeDtypeStruct((B, D), jnp.float32),
        scratch_types=[
            pltpu.VMEM((b_per_w,), jnp.int32),
            pltpu.VMEM((_K, D), jnp.float32),
            pltpu.VMEM((_K, D), jnp.float32),
            pltpu.SemaphoreType.DMA,
            pltpu.SemaphoreType.DMA,
            pltpu.SemaphoreType.DMA,
            pltpu.SemaphoreType.DMA,
        ],
    )
    def gather(tab_hbm, idx_hbm, out_hbm, idx_v, rows0, rows1, gs0, gs1, os0, os1):
        wid = lax.axis_index("s") * _NC + lax.axis_index("c")
        base = wid * b_per_w
        pltpu.sync_copy(idx_hbm.at[pl.ds(base, b_per_w)], idx_v)
        bufs = (rows0, rows1)
        gsems = (gs0, gs1)
        osems = (os0, os1)
        h_in = [None, None]
        h_out = [None, None]
        h_in[0] = pltpu.async_copy(
            tab_hbm.at[idx_v.at[pl.ds(0, _K)]], bufs[0], gsems[0]
        )
        for g in range(nchunk):
            b = g % 2
            if g + 1 < nchunk:
                b2 = (g + 1) % 2
                if h_out[b2] is not None:
                    h_out[b2].wait()
                h_in[b2] = pltpu.async_copy(
                    tab_hbm.at[idx_v.at[pl.ds((g + 1) * _K, _K)]],
                    bufs[b2],
                    gsems[b2],
                )
            h_in[b].wait()
            h_out[b] = pltpu.async_copy(
                bufs[b], out_hbm.at[pl.ds(base + g * _K, _K)], osems[b]
            )
        for h in h_out:
            if h is not None:
                h.wait()

    return gather


def kernel(x, table, gamma, beta):
    batch, hist = x.shape
    B = batch * hist
    table_n = _ln_table(table, gamma, beta)
    idx_lmajor = x.T.reshape(B)
    out_flat = _make_gather(B)(table_n, idx_lmajor)
    return out_flat.reshape(hist, batch, D).transpose(1, 0, 2)


# speedup vs baseline: 7.7542x; 1.0280x over previous
import functools

import jax
import jax.numpy as jnp
from jax import lax
from jax.experimental import pallas as pl
from jax.experimental.pallas import tpu as pltpu
from jax.experimental.pallas import tpu_sc as plsc

EPS = 1e-5
D = 128
_INV_D = 1.0 / D


_ROWS_BLK = 2000


def _ln_body(tab_ref, gamma_ref, beta_ref, out_ref):
    t = tab_ref[...]
    ones_over_d = jnp.full((D, D), _INV_D, dtype=jnp.float32)
    mean = jax.lax.dot(t, ones_over_d)
    m2 = jax.lax.dot(t * t, ones_over_d)
    var = m2 - mean * mean
    out_ref[...] = (t - mean) * lax.rsqrt(var + EPS) * gamma_ref[...] + beta_ref[...]


def _ln_table(table, gamma, beta):
    n_rows = table.shape[0]
    grid = n_rows // _ROWS_BLK
    return pl.pallas_call(
        _ln_body,
        grid=(grid,),
        in_specs=[
            pl.BlockSpec((_ROWS_BLK, D), lambda i: (i, 0)),
            pl.BlockSpec((1, D), lambda i: (0, 0)),
            pl.BlockSpec((1, D), lambda i: (0, 0)),
        ],
        out_specs=pl.BlockSpec((_ROWS_BLK, D), lambda i: (i, 0)),
        out_shape=jax.ShapeDtypeStruct((n_rows, D), jnp.float32),
    )(table, gamma.reshape(1, D), beta.reshape(1, D))


_NC, _NS = 2, 16
_NW = _NC * _NS
_K = 400


def _make_gather(B):
    b_per_w = B // _NW
    nchunk = b_per_w // _K
    mesh = plsc.VectorSubcoreMesh(core_axis_name="c", subcore_axis_name="s")

    @functools.partial(
        pl.kernel,
        mesh=mesh,
        out_type=jax.ShapeDtypeStruct((B, D), jnp.float32),
        scratch_types=[
            pltpu.VMEM((b_per_w,), jnp.int32),
            pltpu.VMEM((_K, D), jnp.float32),
            pltpu.VMEM((_K, D), jnp.float32),
            pltpu.SemaphoreType.DMA,
            pltpu.SemaphoreType.DMA,
            pltpu.SemaphoreType.DMA,
            pltpu.SemaphoreType.DMA,
        ],
    )
    def gather(tab_hbm, idx_hbm, out_hbm, idx_v, rows0, rows1, gs0, gs1, os0, os1):
        wid = lax.axis_index("s") * _NC + lax.axis_index("c")
        base = wid * b_per_w
        pltpu.sync_copy(idx_hbm.at[pl.ds(base, b_per_w)], idx_v)
        bufs = (rows0, rows1)
        gsems = (gs0, gs1)
        osems = (os0, os1)
        h_in = [None, None]
        h_out = [None, None]
        h_in[0] = pltpu.async_copy(
            tab_hbm.at[idx_v.at[pl.ds(0, _K)]], bufs[0], gsems[0]
        )
        for g in range(nchunk):
            b = g % 2
            if g + 1 < nchunk:
                b2 = (g + 1) % 2
                if h_out[b2] is not None:
                    h_out[b2].wait()
                h_in[b2] = pltpu.async_copy(
                    tab_hbm.at[idx_v.at[pl.ds((g + 1) * _K, _K)]],
                    bufs[b2],
                    gsems[b2],
                )
            h_in[b].wait()
            h_out[b] = pltpu.async_copy(
                bufs[b], out_hbm.at[pl.ds(base + g * _K, _K)], osems[b]
            )
        for h in h_out:
            if h is not None:
                h.wait()

    return gather


def kernel(x, table, gamma, beta):
    batch, hist = x.shape
    B = batch * hist
    table_n = _ln_table(table, gamma, beta)
    idx_lmajor = x.T.reshape(B)
    out_flat = _make_gather(B)(table_n, idx_lmajor)
    return out_flat.reshape(hist, batch, D).transpose(1, 0, 2)


# speedup vs baseline: 8.5012x; 1.0963x over previous
import functools

import jax
import jax.numpy as jnp
from jax import lax
from jax.experimental import pallas as pl
from jax.experimental.pallas import tpu as pltpu
from jax.experimental.pallas import tpu_sc as plsc

EPS = 1e-5
D = 128
_INV_D = 1.0 / D


_ROWS_BLK = 5000


def _ln_body(tab_ref, gamma_ref, beta_ref, out_ref):
    t = tab_ref[...]
    ones_over_d = jnp.full((D, D), _INV_D, dtype=jnp.float32)
    mean = jax.lax.dot(t, ones_over_d)
    m2 = jax.lax.dot(t * t, ones_over_d)
    var = m2 - mean * mean
    out_ref[...] = (t - mean) * lax.rsqrt(var + EPS) * gamma_ref[...] + beta_ref[...]


def _ln_table(table, gamma, beta):
    n_rows = table.shape[0]
    grid = n_rows // _ROWS_BLK
    return pl.pallas_call(
        _ln_body,
        grid=(grid,),
        in_specs=[
            pl.BlockSpec((_ROWS_BLK, D), lambda i: (i, 0)),
            pl.BlockSpec((1, D), lambda i: (0, 0)),
            pl.BlockSpec((1, D), lambda i: (0, 0)),
        ],
        out_specs=pl.BlockSpec((_ROWS_BLK, D), lambda i: (i, 0)),
        out_shape=jax.ShapeDtypeStruct((n_rows, D), jnp.float32),
    )(table, gamma.reshape(1, D), beta.reshape(1, D))


_NC, _NS = 2, 16
_NW = _NC * _NS
_K = 400


def _make_gather(B):
    b_per_w = B // _NW
    nchunk = b_per_w // _K
    mesh = plsc.VectorSubcoreMesh(core_axis_name="c", subcore_axis_name="s")

    @functools.partial(
        pl.kernel,
        mesh=mesh,
        out_type=jax.ShapeDtypeStruct((B, D), jnp.float32),
        scratch_types=[
            pltpu.VMEM((b_per_w,), jnp.int32),
            pltpu.VMEM((_K, D), jnp.float32),
            pltpu.VMEM((_K, D), jnp.float32),
            pltpu.SemaphoreType.DMA,
            pltpu.SemaphoreType.DMA,
            pltpu.SemaphoreType.DMA,
            pltpu.SemaphoreType.DMA,
        ],
    )
    def gather(tab_hbm, idx_hbm, out_hbm, idx_v, rows0, rows1, gs0, gs1, os0, os1):
        wid = lax.axis_index("s") * _NC + lax.axis_index("c")
        base = wid * b_per_w
        pltpu.sync_copy(idx_hbm.at[pl.ds(base, b_per_w)], idx_v)
        bufs = (rows0, rows1)
        gsems = (gs0, gs1)
        osems = (os0, os1)
        h_in = [None, None]
        h_out = [None, None]
        h_in[0] = pltpu.async_copy(
            tab_hbm.at[idx_v.at[pl.ds(0, _K)]], bufs[0], gsems[0]
        )
        for g in range(nchunk):
            b = g % 2
            if g + 1 < nchunk:
                b2 = (g + 1) % 2
                if h_out[b2] is not None:
                    h_out[b2].wait()
                h_in[b2] = pltpu.async_copy(
                    tab_hbm.at[idx_v.at[pl.ds((g + 1) * _K, _K)]],
                    bufs[b2],
                    gsems[b2],
                )
            h_in[b].wait()
            h_out[b] = pltpu.async_copy(
                bufs[b], out_hbm.at[pl.ds(base + g * _K, _K)], osems[b]
            )
        for h in h_out:
            if h is not None:
                h.wait()

    return gather


def kernel(x, table, gamma, beta):
    batch, hist = x.shape
    B = batch * hist
    table_n = _ln_table(table, gamma, beta)
    idx_lmajor = x.T.reshape(B)
    out_flat = _make_gather(B)(table_n, idx_lmajor)
    return out_flat.reshape(hist, batch, D).transpose(1, 0, 2)


# speedup vs baseline: 9.0909x; 1.0694x over previous
import functools

import jax
import jax.numpy as jnp
from jax import lax
from jax.experimental import pallas as pl
from jax.experimental.pallas import tpu as pltpu
from jax.experimental.pallas import tpu_sc as plsc

EPS = 1e-5
D = 128
_INV_D = 1.0 / D


_ROWS_BLK = 10000


def _ln_body(tab_ref, gamma_ref, beta_ref, out_ref):
    t = tab_ref[...]
    ones_over_d = jnp.full((D, D), _INV_D, dtype=jnp.float32)
    mean = jax.lax.dot(t, ones_over_d)
    m2 = jax.lax.dot(t * t, ones_over_d)
    var = m2 - mean * mean
    out_ref[...] = (t - mean) * lax.rsqrt(var + EPS) * gamma_ref[...] + beta_ref[...]


def _ln_table(table, gamma, beta):
    n_rows = table.shape[0]
    grid = n_rows // _ROWS_BLK
    return pl.pallas_call(
        _ln_body,
        grid=(grid,),
        in_specs=[
            pl.BlockSpec((_ROWS_BLK, D), lambda i: (i, 0)),
            pl.BlockSpec((1, D), lambda i: (0, 0)),
            pl.BlockSpec((1, D), lambda i: (0, 0)),
        ],
        out_specs=pl.BlockSpec((_ROWS_BLK, D), lambda i: (i, 0)),
        out_shape=jax.ShapeDtypeStruct((n_rows, D), jnp.float32),
    )(table, gamma.reshape(1, D), beta.reshape(1, D))


_NC, _NS = 2, 16
_NW = _NC * _NS
_K = 400


def _make_gather(B):
    b_per_w = B // _NW
    nchunk = b_per_w // _K
    mesh = plsc.VectorSubcoreMesh(core_axis_name="c", subcore_axis_name="s")

    @functools.partial(
        pl.kernel,
        mesh=mesh,
        out_type=jax.ShapeDtypeStruct((B, D), jnp.float32),
        scratch_types=[
            pltpu.VMEM((b_per_w,), jnp.int32),
            pltpu.VMEM((_K, D), jnp.float32),
            pltpu.VMEM((_K, D), jnp.float32),
            pltpu.SemaphoreType.DMA,
            pltpu.SemaphoreType.DMA,
            pltpu.SemaphoreType.DMA,
            pltpu.SemaphoreType.DMA,
        ],
    )
    def gather(tab_hbm, idx_hbm, out_hbm, idx_v, rows0, rows1, gs0, gs1, os0, os1):
        wid = lax.axis_index("s") * _NC + lax.axis_index("c")
        base = wid * b_per_w
        pltpu.sync_copy(idx_hbm.at[pl.ds(base, b_per_w)], idx_v)
        bufs = (rows0, rows1)
        gsems = (gs0, gs1)
        osems = (os0, os1)
        h_in = [None, None]
        h_out = [None, None]
        h_in[0] = pltpu.async_copy(
            tab_hbm.at[idx_v.at[pl.ds(0, _K)]], bufs[0], gsems[0]
        )
        for g in range(nchunk):
            b = g % 2
            if g + 1 < nchunk:
                b2 = (g + 1) % 2
                if h_out[b2] is not None:
                    h_out[b2].wait()
                h_in[b2] = pltpu.async_copy(
                    tab_hbm.at[idx_v.at[pl.ds((g + 1) * _K, _K)]],
                    bufs[b2],
                    gsems[b2],
                )
            h_in[b].wait()
            h_out[b] = pltpu.async_copy(
                bufs[b], out_hbm.at[pl.ds(base + g * _K, _K)], osems[b]
            )
        for h in h_out:
            if h is not None:
                h.wait()

    return gather


def kernel(x, table, gamma, beta):
    batch, hist = x.shape
    B = batch * hist
    table_n = _ln_table(table, gamma, beta)
    idx_lmajor = x.T.reshape(B)
    out_flat = _make_gather(B)(table_n, idx_lmajor)
    return out_flat.reshape(hist, batch, D).transpose(1, 0, 2)
